# Initial kernel scaffold; baseline (speedup 1.0000x reference)
import jax
import jax.numpy as jnp
from jax.experimental import pallas as pl


def kernel(xb, w_pad):
    raise NotImplementedError("write your pallas kernel here")


# speedup vs baseline: 61.0740x; 61.0740x over previous
import jax
import jax.numpy as jnp
from jax.experimental import pallas as pl
from jax.experimental.pallas import tpu as pltpu

_IN_FEATURES = 28 * 28
_OUT_FEATURES = 5
_TAU = 2.0
_V_THRESHOLD = 0.8
_M_PAD = 8

_BM = 1024


def _vpr_block_kernel(x_ref, w_ref, cur_ref, spk_ref):
    x = x_ref[...]
    w = w_ref[...]
    cur = jax.lax.dot_general(
        w, x,
        dimension_numbers=(((1,), (1,)), ((), ())),
        preferred_element_type=jnp.float32,
    )
    cur_ref[...] = cur
    spk_ref[...] = (cur / _TAU >= _V_THRESHOLD).astype(jnp.float32)


@jax.jit
def kernel(xb, w_pad):
    b = xb.shape[0]
    xb_flat = jnp.reshape(xb, (b, _IN_FEATURES)).astype(jnp.float32)

    bm = _BM if b >= _BM else max(8, b)
    nb = pl.cdiv(b, bm)
    b_pad = nb * bm
    if b_pad != b:
        xb_flat = jnp.pad(xb_flat, ((0, b_pad - b), (0, 0)))

    cur_t, spk_t = pl.pallas_call(
        _vpr_block_kernel,
        out_shape=(
            jax.ShapeDtypeStruct((_M_PAD, b_pad), jnp.float32),
            jax.ShapeDtypeStruct((_M_PAD, b_pad), jnp.float32),
        ),
        grid=(nb,),
        in_specs=[
            pl.BlockSpec((bm, _IN_FEATURES), lambda i: (i, 0)),
            pl.BlockSpec((_M_PAD, _IN_FEATURES), lambda i: (0, 0)),
        ],
        out_specs=(
            pl.BlockSpec((_M_PAD, bm), lambda i: (0, i)),
            pl.BlockSpec((_M_PAD, bm), lambda i: (0, i)),
        ),
        compiler_params=pltpu.CompilerParams(
            dimension_semantics=("parallel",)),
    )(xb_flat, w_pad.astype(jnp.float32))

    cur = cur_t[:_OUT_FEATURES, :b].T
    spikes = spk_t[:_OUT_FEATURES, :b].T
    return spikes, cur


# speedup vs baseline: 62.4508x; 1.0225x over previous
import jax
import jax.numpy as jnp
from jax.experimental import pallas as pl
from jax.experimental.pallas import tpu as pltpu

_IN_FEATURES = 28 * 28
_OUT_FEATURES = 5
_TAU = 2.0
_V_THRESHOLD = 0.8
_M_PAD = 8

_BM = 2048


def _vpr_block_kernel(x_ref, w_ref, cur_ref, spk_ref):
    x = x_ref[...]
    w = w_ref[...]
    cur = jax.lax.dot_general(
        w, x,
        dimension_numbers=(((1,), (1,)), ((), ())),
        preferred_element_type=jnp.float32,
    )
    cur_ref[...] = cur
    spk_ref[...] = (cur / _TAU >= _V_THRESHOLD).astype(jnp.float32)


@jax.jit
def kernel(xb, w_pad):
    b = xb.shape[0]
    xb_flat = jnp.reshape(xb, (b, _IN_FEATURES)).astype(jnp.float32)

    bm = _BM if b >= _BM else max(8, b)
    nb = pl.cdiv(b, bm)
    b_pad = nb * bm
    if b_pad != b:
        xb_flat = jnp.pad(xb_flat, ((0, b_pad - b), (0, 0)))

    cur_t, spk_t = pl.pallas_call(
        _vpr_block_kernel,
        out_shape=(
            jax.ShapeDtypeStruct((_M_PAD, b_pad), jnp.float32),
            jax.ShapeDtypeStruct((_M_PAD, b_pad), jnp.float32),
        ),
        grid=(nb,),
        in_specs=[
            pl.BlockSpec((bm, _IN_FEATURES), lambda i: (i, 0)),
            pl.BlockSpec((_M_PAD, _IN_FEATURES), lambda i: (0, 0)),
        ],
        out_specs=(
            pl.BlockSpec((_M_PAD, bm), lambda i: (0, i)),
            pl.BlockSpec((_M_PAD, bm), lambda i: (0, i)),
        ),
        compiler_params=pltpu.CompilerParams(
            dimension_semantics=("parallel",)),
    )(xb_flat, w_pad.astype(jnp.float32))

    cur = cur_t[:_OUT_FEATURES, :b].T
    spikes = spk_t[:_OUT_FEATURES, :b].T
    return spikes, cur
